# baseline (speedup 1.0000x reference)
import jax
import jax.numpy as jnp
from jax.experimental import pallas as pl
from jax.experimental.pallas import tpu as pltpu

NTOK = 16384
DIM = 4096
NE = 64
BT = 512
NHALF = NTOK // 2
HGRID = NHALF // BT


def _route_block(x_ref, wt_ref, b_ref):
    logits = jnp.dot(x_ref[...], wt_ref[...], preferred_element_type=jnp.float32)
    logits = logits + b_ref[...]

    m = jnp.max(logits, axis=1, keepdims=True)
    e = jnp.exp(logits - m)
    scores = e / jnp.sum(e, axis=1, keepdims=True)

    iota = jax.lax.broadcasted_iota(jnp.int32, scores.shape, 1)
    key = (scores.view(jnp.int32) & jnp.int32(~0x3F)) | (jnp.int32(NE - 1) - iota)
    mask = None
    out1 = None
    for k in range(4):
        mx = jnp.max(key, axis=1, keepdims=True)
        sel = key == mx
        if k == 0:
            out1 = jnp.where(sel, scores, 0.0)
            mask = sel
        else:
            mask = jnp.logical_or(mask, sel)
        key = jnp.where(sel, jnp.int32(-2147483648), key)

    out4 = jnp.where(mask, scores, 0.0)
    ssum = jnp.sum(scores, axis=0, keepdims=True)
    msum = jnp.sum(mask.astype(jnp.float32), axis=0, keepdims=True)
    return out4, out1, ssum, msum


def _gate_kernel(x1_ref, x2_ref, wt_ref, b_ref, out4_ref, out1_ref, sums_ref):
    for h, x_ref in enumerate((x1_ref, x2_ref)):
        out4, out1, ssum, msum = _route_block(x_ref, wt_ref, b_ref)
        out4_ref[h] = out4
        out1_ref[h] = out1
        sums_ref[h, 0, 0:1, :] = ssum
        sums_ref[h, 0, 1:2, :] = msum


def _loss_kernel(sums_ref, loss_ref):
    ssum = jnp.sum(sums_ref[:, :, 0, :], axis=(0, 1), keepdims=True)[:, 0]
    msum = jnp.sum(sums_ref[:, :, 1, :], axis=(0, 1), keepdims=True)[:, 0]
    n = jnp.float32(NTOK)
    loss_ref[...] = NE * jnp.sum(ssum * msum, axis=1, keepdims=True) / (n * n)


@jax.jit
def _gate(x, wt, b2):
    out4, out1, sums = pl.pallas_call(
        _gate_kernel,
        grid=(HGRID,),
        in_specs=[
            pl.BlockSpec((BT, DIM), lambda i: (i, 0)),
            pl.BlockSpec((BT, DIM), lambda i: (i + HGRID, 0)),
            pl.BlockSpec((DIM, NE), lambda i: (0, 0)),
            pl.BlockSpec((1, NE), lambda i: (0, 0)),
        ],
        out_specs=[
            pl.BlockSpec((2, BT, NE), lambda i: (0, i, 0)),
            pl.BlockSpec((2, BT, NE), lambda i: (0, i, 0)),
            pl.BlockSpec((2, 1, 2, NE), lambda i: (0, i, 0, 0)),
        ],
        out_shape=[
            jax.ShapeDtypeStruct((2, NHALF, NE), jnp.float32),
            jax.ShapeDtypeStruct((2, NHALF, NE), jnp.float32),
            jax.ShapeDtypeStruct((2, HGRID, 2, NE), jnp.float32),
        ],
        compiler_params=pltpu.CompilerParams(
            dimension_semantics=("parallel",),
        ),
    )(x, x, wt, b2)
    loss = pl.pallas_call(
        _loss_kernel,
        out_shape=jax.ShapeDtypeStruct((1, 1), jnp.float32),
    )(sums)
    return (out4.reshape(NTOK, NE), loss.reshape(()), out1.reshape(NTOK, NE))


def kernel(x, W, b):
    return _gate(x, W.T, b.reshape(1, NE))


# speedup vs baseline: 1.0030x; 1.0030x over previous
import functools

import jax
import jax.numpy as jnp
from jax.experimental import pallas as pl
from jax.experimental.pallas import tpu as pltpu

NTOK = 16384
DIM = 4096
NE = 64
BT = 1024
NSTEPS = NTOK // BT


def _gate_kernel(x_ref, wt_ref, b_ref, out4_ref, out1_ref, sums_ref):
    logits = jnp.dot(x_ref[...], wt_ref[...], preferred_element_type=jnp.float32)
    logits = logits + b_ref[...]

    m = jnp.max(logits, axis=1, keepdims=True)
    e = jnp.exp(logits - m)
    scores = e / jnp.sum(e, axis=1, keepdims=True)

    iota = jax.lax.broadcasted_iota(jnp.int32, scores.shape, 1)
    cur = scores
    mask = None
    for k in range(4):
        mx = jnp.max(cur, axis=1, keepdims=True)
        first = jnp.min(jnp.where(cur == mx, iota, NE), axis=1, keepdims=True)
        sel = iota == first
        if k == 0:
            out1_ref[...] = jnp.where(sel, scores, 0.0)
            mask = sel
        else:
            mask = jnp.logical_or(mask, sel)
        cur = jnp.where(sel, -jnp.inf, cur)

    out4_ref[...] = jnp.where(mask, scores, 0.0)

    sums_ref[0, 0:1, :] = jnp.sum(scores, axis=0, keepdims=True)
    sums_ref[0, 1:2, :] = jnp.sum(mask.astype(jnp.float32), axis=0, keepdims=True)


def _loss_kernel(sums_ref, loss_ref):
    ssum = jnp.sum(sums_ref[:, 0, :], axis=0, keepdims=True)
    msum = jnp.sum(sums_ref[:, 1, :], axis=0, keepdims=True)
    n = jnp.float32(NTOK)
    loss_ref[...] = NE * jnp.sum(ssum * msum, axis=1, keepdims=True) / (n * n)


@jax.jit
def _gate(x, wt, b2):
    out4, out1, sums = pl.pallas_call(
        _gate_kernel,
        grid=(NSTEPS,),
        in_specs=[
            pl.BlockSpec((BT, DIM), lambda i: (i, 0)),
            pl.BlockSpec((DIM, NE), lambda i: (0, 0)),
            pl.BlockSpec((1, NE), lambda i: (0, 0)),
        ],
        out_specs=[
            pl.BlockSpec((BT, NE), lambda i: (i, 0)),
            pl.BlockSpec((BT, NE), lambda i: (i, 0)),
            pl.BlockSpec((1, 2, NE), lambda i: (i, 0, 0)),
        ],
        out_shape=[
            jax.ShapeDtypeStruct((NTOK, NE), jnp.float32),
            jax.ShapeDtypeStruct((NTOK, NE), jnp.float32),
            jax.ShapeDtypeStruct((NSTEPS, 2, NE), jnp.float32),
        ],
        compiler_params=pltpu.CompilerParams(
            dimension_semantics=("parallel",),
        ),
    )(x, wt, b2)
    loss = pl.pallas_call(
        _loss_kernel,
        out_shape=jax.ShapeDtypeStruct((1, 1), jnp.float32),
    )(sums)
    return out4, loss.reshape(()), out1


def kernel(x, W, b):
    return _gate(x, W.T, b.reshape(1, NE))
